# baseline (speedup 1.0000x reference)
import functools

import jax
import jax.numpy as jnp
from jax import lax
from jax.experimental import pallas as pl
from jax.experimental.pallas import tpu as pltpu
from jax.experimental.pallas import tpu_sc as plsc

N_DST = 10000
N_SRC = 10000
E = 320000
D = 128

L = 16
N_TILES = 16
EDGES_PER_TILE = E // N_TILES
CHUNK = 80
N_CHUNKS = EDGES_PER_TILE // CHUNK
ROWS_PER_TILE = 640
N_PAD = N_TILES * ROWS_PER_TILE

BLK = 2048


def _logits_body(fd, fs0, fs1, al0, ar0, al1, ar1,
                 el0_o, er0_o, el1_o, er1_o, mx_o):
    el0 = jnp.sum(fs0[...] * al0[...], axis=1, keepdims=True)
    er0 = jnp.sum(fd[...] * ar0[...], axis=1, keepdims=True)
    el1 = jnp.sum(fs1[...] * al1[...], axis=1, keepdims=True)
    er1 = jnp.sum(fd[...] * ar1[...], axis=1, keepdims=True)
    el0_o[...] = el0
    er0_o[...] = er0
    el1_o[...] = el1
    er1_o[...] = er1
    mx = jnp.stack([jnp.max(el0), jnp.max(el1)])
    mx_o[...] = jnp.broadcast_to(mx[:, None], (2, 128))


def _logits_call(fd, fs0, fs1, al0, ar0, al1, ar1):
    vec = jax.ShapeDtypeStruct((N_SRC, 1), jnp.float32)
    mxs = jax.ShapeDtypeStruct((2, 128), jnp.float32)
    return pl.pallas_call(
        _logits_body,
        out_shape=(vec, vec, vec, vec, mxs),
        compiler_params=pltpu.CompilerParams(vmem_limit_bytes=100 * 1024 * 1024),
    )(fd, fs0, fs1, al0, ar0, al1, ar1)


G = 10
N_GROUPS = N_CHUNKS // G
GEDGES = G * CHUNK
SUB = 40
N_SUBS = EDGES_PER_TILE // SUB
SUBS_PER_GROUP = GEDGES // SUB
PAIRS = SUBS_PER_GROUP // 2


def _sc_body(fs0, fs1, src0, dst0t, src1, dst1t,
             el0, er0, el1, er1, mx,
             out0, out1, den0, den1,
             el_b, er_b, mx_b, den_b, src_f, dstb, wbuf, rowsA, rowsB,
             acc_sh, gsemA, gsemB, ssemA, ssemB):
    g = lax.axis_index("c")
    t = lax.axis_index("s")

    zero16 = jnp.zeros((L,), jnp.float32)
    lane = lax.iota(jnp.int32, L)
    cols = [lane + d * L for d in range(D // L)]

    @pl.loop(0, N_PAD // L)
    def _(i):
        den_b[pl.ds(i * L, L)] = zero16

    @pl.loop(0, SUB)
    def _(b):
        bsp = jnp.full((L,), b, jnp.int32)
        for d in range(D // L):
            plsc.store_scatter(rowsA, [bsp, cols[d]], zero16)

    for kk in range(ROWS_PER_TILE // SUB):
        pltpu.sync_copy(
            rowsA, acc_sh.at[pl.ds(t * ROWS_PER_TILE + kk * SUB, SUB), :])
    plsc.subcore_barrier()

    def run_graph(gi, fs, srcr, dstrt, el, er, out_o, den_o):
        pltpu.sync_copy(el, el_b)
        pltpu.sync_copy(er, er_b)
        pltpu.sync_copy(mx.at[gi], mx_b)
        msp = jnp.full((L,), mx_b[pl.ds(0, L)][0], jnp.float32)

        rowoffs = []
        colks = []
        for k in range(2 * SUB // L):
            q = lane + k * L
            ro = jnp.where(q >= SUB, 1, 0).astype(jnp.int32)
            rowoffs.append(ro)
            colks.append(q - ro * SUB)

        def scale(buf, off):
            def do_edges(g, n):
                wv = wbuf[pl.ds(off + g * L, L)]
                for i in range(n):
                    spl = wv[jnp.full((L,), i, jnp.int32)]
                    bsp = jnp.full((L,), g * L + i, jnp.int32)
                    for d in range(D // L):
                        v = plsc.load_gather(buf, [bsp, cols[d]])
                        plsc.store_scatter(buf, [bsp, cols[d]], v * spl)

            @pl.loop(0, SUB // L)
            def _(g):
                do_edges(g, L)
            do_edges(SUB // L, SUB - SUB // L * L)

        def gather_start(c, buf, sem_):
            pltpu.async_copy(fs.at[src_f.at[pl.ds(c * SUB, SUB)]], buf, sem_)

        def gather_wait(c, buf, sem_):
            pltpu.make_async_copy(fs.at[src_f.at[pl.ds(c * SUB, SUB)]],
                                  buf, sem_).wait()

        def scat_start(c, buf, sem_):
            pltpu.async_copy(buf, acc_sh.at[dstb.at[c]], sem_, add=True)

        def scat_wait(c, buf, sem_):
            pltpu.make_async_copy(buf, acc_sh.at[dstb.at[c]], sem_).wait()

        @pl.loop(0, N_GROUPS)
        def _(grp):
            pltpu.sync_copy(srcr.at[t, pl.ds(grp * GEDGES, GEDGES)], src_f)
            pltpu.sync_copy(
                dstrt.at[t, pl.ds(grp * SUBS_PER_GROUP, SUBS_PER_GROUP), :],
                dstb)
            gather_start(0, rowsA, gsemA)

            @pl.loop(0, PAIRS)
            def _(p):
                c0 = 2 * p
                c1 = c0 + 1
                base = c0 * SUB
                for k in range(2 * SUB // L):
                    sidx = src_f[pl.ds(base + k * L, L)]
                    didx = plsc.load_gather(
                        dstb, [jnp.full((L,), c0, jnp.int32) + rowoffs[k],
                               colks[k]])
                    elv = plsc.load_gather(el_b, [sidx])
                    erv = plsc.load_gather(er_b, [didx])
                    x = elv + erv
                    e = jnp.maximum(x, 0.01 * x)
                    x2 = erv + msp
                    cv = jnp.maximum(x2, 0.01 * x2)
                    w = jnp.exp(e - cv)
                    wbuf[pl.ds(k * L, L)] = w
                    plsc.addupdate_scatter(den_b, [didx], w)

                @pl.when(p > 0)
                def _():
                    scat_wait(c1 - 2, rowsB, ssemB)
                gather_start(c1, rowsB, gsemB)
                gather_wait(c0, rowsA, gsemA)
                scale(rowsA, 0)
                scat_start(c0, rowsA, ssemA)
                gather_wait(c1, rowsB, gsemB)
                scale(rowsB, SUB)
                scat_start(c1, rowsB, ssemB)

                @pl.when(p < PAIRS - 1)
                def _():
                    scat_wait(c0, rowsA, ssemA)
                    gather_start(c0 + 2, rowsA, gsemA)

            scat_wait(SUBS_PER_GROUP - 2, rowsA, ssemA)
            scat_wait(SUBS_PER_GROUP - 1, rowsB, ssemB)

        pltpu.sync_copy(den_b, den_o.at[t])
        plsc.subcore_barrier()
        pltpu.sync_copy(acc_sh.at[pl.ds(t * ROWS_PER_TILE, ROWS_PER_TILE), :],
                        out_o.at[pl.ds(t * ROWS_PER_TILE, ROWS_PER_TILE), :])

    @pl.when(g == 0)
    def _():
        run_graph(0, fs0, src0, dst0t, el0, er0, out0, den0)

    @pl.when(g == 1)
    def _():
        run_graph(1, fs1, src1, dst1t, el1, er1, out1, den1)


def _sc_call(fs0, fs1, src0, dst0t, src1, dst1t,
             el0, er0, el1, er1, mx):
    mesh = plsc.VectorSubcoreMesh(core_axis_name="c", subcore_axis_name="s")
    mat = jax.ShapeDtypeStruct((N_PAD, D), jnp.float32)
    vec = jax.ShapeDtypeStruct((N_TILES, N_PAD), jnp.float32)
    kern = pl.kernel(
        _sc_body,
        out_type=(mat, mat, vec, vec),
        mesh=mesh,
        compiler_params=pltpu.CompilerParams(needs_layout_passes=False,
                                             use_tc_tiling_on_sc=False),
        scratch_types=[
            pltpu.VMEM((N_SRC,), jnp.float32),
            pltpu.VMEM((N_DST,), jnp.float32),
            pltpu.VMEM((128,), jnp.float32),
            pltpu.VMEM((N_PAD,), jnp.float32),
            pltpu.VMEM((GEDGES,), jnp.int32),
            pltpu.VMEM((SUBS_PER_GROUP, SUB), jnp.int32),
            pltpu.VMEM((CHUNK + L,), jnp.float32),
            pltpu.VMEM((SUB, D), jnp.float32),
            pltpu.VMEM((SUB, D), jnp.float32),
            pltpu.VMEM_SHARED((N_PAD, D), jnp.float32),
            pltpu.SemaphoreType.DMA,
            pltpu.SemaphoreType.DMA,
            pltpu.SemaphoreType.DMA,
            pltpu.SemaphoreType.DMA,
        ],
    )
    return kern(fs0, fs1, src0, dst0t, src1, dst1t,
                el0, er0, el1, er1, mx)


def _combine1_body(o0, d0, o1, d1, W, b, h0_o, h1_o, s0_o, s1_o):
    i = pl.program_id(0)

    def head(o, dd):
        dv = jnp.maximum(jnp.sum(dd[...], axis=0), 1e-9)[:, None]
        h = o[...] / dv
        return jnp.where(h > 0, h, jnp.exp(h) - 1.0)

    h0 = head(o0, d0)
    h1 = head(o1, d1)
    h0_o[...] = h0
    h1_o[...] = h1

    rowid = lax.broadcasted_iota(jnp.int32, (BLK, 1), 0) + i * BLK
    msk = (rowid < N_DST).astype(jnp.float32)
    Wt = W[...].T

    def tsum(h):
        tv = jnp.tanh(jnp.dot(h, Wt, preferred_element_type=jnp.float32)
                      + b[...])
        return jnp.broadcast_to(jnp.sum(tv * msk, axis=0)[None, :], (8, D))

    a0 = tsum(h0)
    a1 = tsum(h1)

    @pl.when(i == 0)
    def _():
        s0_o[...] = a0
        s1_o[...] = a1

    @pl.when(i > 0)
    def _():
        s0_o[...] = s0_o[...] + a0
        s1_o[...] = s1_o[...] + a1


def _combine1_call(o0, d0, o1, d1, W, b):
    nblk = N_PAD // BLK
    mat_spec = pl.BlockSpec((BLK, D), lambda i: (i, 0))
    vec_spec = pl.BlockSpec((N_TILES, BLK), lambda i: (0, i))
    full = pl.BlockSpec((D, D), lambda i: (0, 0))
    brow = pl.BlockSpec((1, D), lambda i: (0, 0))
    s_spec = pl.BlockSpec((8, D), lambda i: (0, 0))
    return pl.pallas_call(
        _combine1_body,
        grid=(nblk,),
        in_specs=[mat_spec, vec_spec, mat_spec, vec_spec, full, brow],
        out_specs=(mat_spec, mat_spec, s_spec, s_spec),
        out_shape=(
            jax.ShapeDtypeStruct((N_PAD, D), jnp.float32),
            jax.ShapeDtypeStruct((N_PAD, D), jnp.float32),
            jax.ShapeDtypeStruct((8, D), jnp.float32),
            jax.ShapeDtypeStruct((8, D), jnp.float32),
        ),
    )(o0, d0, o1, d1, W, b)


def _combine2_body(beta, h0, h1, z_o):
    z_o[...] = h0[...] * beta[0, 0] + h1[...] * beta[0, 1]


def _combine2_call(h0, h1, beta):
    nblk = N_PAD // BLK
    mat_spec = pl.BlockSpec((BLK, D), lambda i: (i, 0))
    return pl.pallas_call(
        _combine2_body,
        grid=(nblk,),
        in_specs=[pl.BlockSpec(memory_space=pltpu.SMEM), mat_spec, mat_spec],
        out_specs=mat_spec,
        out_shape=jax.ShapeDtypeStruct((N_PAD, D), jnp.float32),
    )(beta, h0, h1)


def kernel(feat_dst, feat_src0, feat_src1, edge_index0, edge_index1,
           attn_l0, attn_r0, attn_l1, attn_r1, fc_W, fc_b, sem_attn):
    el0, er0, el1, er1, mx = _logits_call(
        feat_dst, feat_src0, feat_src1, attn_l0, attn_r0, attn_l1, attn_r1)

    src0 = edge_index0[0].reshape(N_TILES, EDGES_PER_TILE)
    dst0t = edge_index0[1].reshape(N_TILES, N_SUBS, SUB)
    src1 = edge_index1[0].reshape(N_TILES, EDGES_PER_TILE)
    dst1t = edge_index1[1].reshape(N_TILES, N_SUBS, SUB)

    out0, out1, den0, den1 = _sc_call(
        feat_src0, feat_src1, src0, dst0t, src1, dst1t,
        el0.reshape(-1), er0.reshape(-1), el1.reshape(-1), er1.reshape(-1),
        mx)

    h0, h1, s0a, s1a = _combine1_call(
        out0, den0, out1, den1, fc_W, fc_b.reshape(1, D))

    mean0 = s0a[0] / N_DST
    mean1 = s1a[0] / N_DST
    wv = jnp.stack([jnp.dot(mean0, sem_attn[0]), jnp.dot(mean1, sem_attn[0])])
    beta = jax.nn.softmax(wv)

    z = _combine2_call(h0, h1, beta.reshape(1, 2))
    return z[:N_DST]


# speedup vs baseline: 1.1412x; 1.1412x over previous
import functools

import jax
import jax.numpy as jnp
from jax import lax
from jax.experimental import pallas as pl
from jax.experimental.pallas import tpu as pltpu
from jax.experimental.pallas import tpu_sc as plsc

N_DST = 10000
N_SRC = 10000
E = 320000
D = 128

L = 16
N_TILES = 16
EDGES_PER_TILE = E // N_TILES
CHUNK = 80
N_CHUNKS = EDGES_PER_TILE // CHUNK
ROWS_PER_TILE = 640
N_PAD = N_TILES * ROWS_PER_TILE

BLK = 2048


def _logits_body(fd, fs0, fs1, al0, ar0, al1, ar1,
                 el0_o, er0_o, el1_o, er1_o, mx_o):
    el0 = jnp.sum(fs0[...] * al0[...], axis=1, keepdims=True)
    er0 = jnp.sum(fd[...] * ar0[...], axis=1, keepdims=True)
    el1 = jnp.sum(fs1[...] * al1[...], axis=1, keepdims=True)
    er1 = jnp.sum(fd[...] * ar1[...], axis=1, keepdims=True)
    el0_o[...] = el0
    er0_o[...] = er0
    el1_o[...] = el1
    er1_o[...] = er1
    mx = jnp.stack([jnp.max(el0), jnp.max(el1)])
    mx_o[...] = jnp.broadcast_to(mx[:, None], (2, 128))


def _logits_call(fd, fs0, fs1, al0, ar0, al1, ar1):
    vec = jax.ShapeDtypeStruct((N_SRC, 1), jnp.float32)
    mxs = jax.ShapeDtypeStruct((2, 128), jnp.float32)
    return pl.pallas_call(
        _logits_body,
        out_shape=(vec, vec, vec, vec, mxs),
        compiler_params=pltpu.CompilerParams(vmem_limit_bytes=100 * 1024 * 1024),
    )(fd, fs0, fs1, al0, ar0, al1, ar1)


G = 10
N_GROUPS = N_CHUNKS // G
GEDGES = G * CHUNK
SUB = 40
N_SUBS = EDGES_PER_TILE // SUB
SUBS_PER_GROUP = GEDGES // SUB
PAIRS = SUBS_PER_GROUP // 2


def _sc_body(fs0, fs1, src0, dst0t, src1, dst1t,
             el0, er0, el1, er1, mx,
             out0, out1, den0, den1,
             el_b, er_b, mx_b, den_b, src_f, dstb, wbuf, rowsA, rowsB,
             acc_sh, gsemA, gsemB, ssemA, ssemB):
    g = lax.axis_index("c")
    t = lax.axis_index("s")

    zero16 = jnp.zeros((L,), jnp.float32)
    lane = lax.iota(jnp.int32, L)
    cols = [lane + d * L for d in range(D // L)]

    @pl.loop(0, N_PAD // L)
    def _(i):
        den_b[pl.ds(i * L, L)] = zero16

    @pl.loop(0, SUB)
    def _(b):
        bsp = jnp.full((L,), b, jnp.int32)
        for d in range(D // L):
            plsc.store_scatter(rowsA, [bsp, cols[d]], zero16)

    for kk in range(ROWS_PER_TILE // SUB):
        pltpu.sync_copy(
            rowsA, acc_sh.at[pl.ds(t * ROWS_PER_TILE + kk * SUB, SUB), :])
    plsc.subcore_barrier()

    def run_graph(gi, fs, srcr, dstrt, el, er, out_o, den_o):
        pltpu.sync_copy(el, el_b)
        pltpu.sync_copy(er, er_b)
        pltpu.sync_copy(mx.at[gi], mx_b)
        msp = jnp.full((L,), mx_b[pl.ds(0, L)][0], jnp.float32)

        rowoffs = []
        colks = []
        for k in range(2 * SUB // L):
            q = lane + k * L
            ro = jnp.where(q >= SUB, 1, 0).astype(jnp.int32)
            rowoffs.append(ro)
            colks.append(q - ro * SUB)

        def scale(buf, off):
            @pl.loop(0, SUB, unroll=8)
            def _(b):
                ws = wbuf[pl.ds(b + off, L)][0]
                spl = jnp.full((L,), ws, jnp.float32)
                bsp = jnp.full((L,), b, jnp.int32)
                for d in range(D // L):
                    v = plsc.load_gather(buf, [bsp, cols[d]])
                    plsc.store_scatter(buf, [bsp, cols[d]], v * spl)

        def gather_start(c, buf, sem_):
            pltpu.async_copy(fs.at[src_f.at[pl.ds(c * SUB, SUB)]], buf, sem_)

        def gather_wait(c, buf, sem_):
            pltpu.make_async_copy(fs.at[src_f.at[pl.ds(c * SUB, SUB)]],
                                  buf, sem_).wait()

        def scat_start(c, buf, sem_):
            pltpu.async_copy(buf, acc_sh.at[dstb.at[c]], sem_, add=True)

        def scat_wait(c, buf, sem_):
            pltpu.make_async_copy(buf, acc_sh.at[dstb.at[c]], sem_).wait()

        @pl.loop(0, N_GROUPS)
        def _(grp):
            pltpu.sync_copy(srcr.at[t, pl.ds(grp * GEDGES, GEDGES)], src_f)
            pltpu.sync_copy(
                dstrt.at[t, pl.ds(grp * SUBS_PER_GROUP, SUBS_PER_GROUP), :],
                dstb)
            gather_start(0, rowsA, gsemA)

            @pl.loop(0, PAIRS)
            def _(p):
                c0 = 2 * p
                c1 = c0 + 1
                base = c0 * SUB
                for k in range(2 * SUB // L):
                    sidx = src_f[pl.ds(base + k * L, L)]
                    didx = plsc.load_gather(
                        dstb, [jnp.full((L,), c0, jnp.int32) + rowoffs[k],
                               colks[k]])
                    elv = plsc.load_gather(el_b, [sidx])
                    erv = plsc.load_gather(er_b, [didx])
                    x = elv + erv
                    e = jnp.maximum(x, 0.01 * x)
                    x2 = erv + msp
                    cv = jnp.maximum(x2, 0.01 * x2)
                    w = jnp.exp(e - cv)
                    wbuf[pl.ds(k * L, L)] = w
                    plsc.addupdate_scatter(den_b, [didx], w)

                @pl.when(p > 0)
                def _():
                    scat_wait(c1 - 2, rowsB, ssemB)
                gather_start(c1, rowsB, gsemB)
                gather_wait(c0, rowsA, gsemA)
                scale(rowsA, 0)
                scat_start(c0, rowsA, ssemA)
                gather_wait(c1, rowsB, gsemB)
                scale(rowsB, SUB)
                scat_start(c1, rowsB, ssemB)

                @pl.when(p < PAIRS - 1)
                def _():
                    scat_wait(c0, rowsA, ssemA)
                    gather_start(c0 + 2, rowsA, gsemA)

            scat_wait(SUBS_PER_GROUP - 2, rowsA, ssemA)
            scat_wait(SUBS_PER_GROUP - 1, rowsB, ssemB)

        pltpu.sync_copy(den_b, den_o.at[t])
        plsc.subcore_barrier()
        pltpu.sync_copy(acc_sh.at[pl.ds(t * ROWS_PER_TILE, ROWS_PER_TILE), :],
                        out_o.at[pl.ds(t * ROWS_PER_TILE, ROWS_PER_TILE), :])

    @pl.when(g == 0)
    def _():
        run_graph(0, fs0, src0, dst0t, el0, er0, out0, den0)

    @pl.when(g == 1)
    def _():
        run_graph(1, fs1, src1, dst1t, el1, er1, out1, den1)


def _sc_call(fs0, fs1, src0, dst0t, src1, dst1t,
             el0, er0, el1, er1, mx):
    mesh = plsc.VectorSubcoreMesh(core_axis_name="c", subcore_axis_name="s")
    mat = jax.ShapeDtypeStruct((N_PAD, D), jnp.float32)
    vec = jax.ShapeDtypeStruct((N_TILES, N_PAD), jnp.float32)
    kern = pl.kernel(
        _sc_body,
        out_type=(mat, mat, vec, vec),
        mesh=mesh,
        compiler_params=pltpu.CompilerParams(needs_layout_passes=False,
                                             use_tc_tiling_on_sc=False),
        scratch_types=[
            pltpu.VMEM((N_SRC,), jnp.float32),
            pltpu.VMEM((N_DST,), jnp.float32),
            pltpu.VMEM((128,), jnp.float32),
            pltpu.VMEM((N_PAD,), jnp.float32),
            pltpu.VMEM((GEDGES,), jnp.int32),
            pltpu.VMEM((SUBS_PER_GROUP, SUB), jnp.int32),
            pltpu.VMEM((CHUNK + L,), jnp.float32),
            pltpu.VMEM((SUB, D), jnp.float32),
            pltpu.VMEM((SUB, D), jnp.float32),
            pltpu.VMEM_SHARED((N_PAD, D), jnp.float32),
            pltpu.SemaphoreType.DMA,
            pltpu.SemaphoreType.DMA,
            pltpu.SemaphoreType.DMA,
            pltpu.SemaphoreType.DMA,
        ],
    )
    return kern(fs0, fs1, src0, dst0t, src1, dst1t,
                el0, er0, el1, er1, mx)


def _combine1_body(o0, d0, o1, d1, W, b, h0_o, h1_o, s0_o, s1_o):
    i = pl.program_id(0)

    def head(o, dd):
        dv = jnp.maximum(jnp.sum(dd[...], axis=0), 1e-9)[:, None]
        h = o[...] / dv
        return jnp.where(h > 0, h, jnp.exp(h) - 1.0)

    h0 = head(o0, d0)
    h1 = head(o1, d1)
    h0_o[...] = h0
    h1_o[...] = h1

    rowid = lax.broadcasted_iota(jnp.int32, (BLK, 1), 0) + i * BLK
    msk = (rowid < N_DST).astype(jnp.float32)
    Wt = W[...].T

    def tsum(h):
        tv = jnp.tanh(jnp.dot(h, Wt, preferred_element_type=jnp.float32)
                      + b[...])
        return jnp.broadcast_to(jnp.sum(tv * msk, axis=0)[None, :], (8, D))

    a0 = tsum(h0)
    a1 = tsum(h1)

    @pl.when(i == 0)
    def _():
        s0_o[...] = a0
        s1_o[...] = a1

    @pl.when(i > 0)
    def _():
        s0_o[...] = s0_o[...] + a0
        s1_o[...] = s1_o[...] + a1


def _combine1_call(o0, d0, o1, d1, W, b):
    nblk = N_PAD // BLK
    mat_spec = pl.BlockSpec((BLK, D), lambda i: (i, 0))
    vec_spec = pl.BlockSpec((N_TILES, BLK), lambda i: (0, i))
    full = pl.BlockSpec((D, D), lambda i: (0, 0))
    brow = pl.BlockSpec((1, D), lambda i: (0, 0))
    s_spec = pl.BlockSpec((8, D), lambda i: (0, 0))
    return pl.pallas_call(
        _combine1_body,
        grid=(nblk,),
        in_specs=[mat_spec, vec_spec, mat_spec, vec_spec, full, brow],
        out_specs=(mat_spec, mat_spec, s_spec, s_spec),
        out_shape=(
            jax.ShapeDtypeStruct((N_PAD, D), jnp.float32),
            jax.ShapeDtypeStruct((N_PAD, D), jnp.float32),
            jax.ShapeDtypeStruct((8, D), jnp.float32),
            jax.ShapeDtypeStruct((8, D), jnp.float32),
        ),
    )(o0, d0, o1, d1, W, b)


def _combine2_body(beta, h0, h1, z_o):
    z_o[...] = h0[...] * beta[0, 0] + h1[...] * beta[0, 1]


def _combine2_call(h0, h1, beta):
    nblk = N_PAD // BLK
    mat_spec = pl.BlockSpec((BLK, D), lambda i: (i, 0))
    return pl.pallas_call(
        _combine2_body,
        grid=(nblk,),
        in_specs=[pl.BlockSpec(memory_space=pltpu.SMEM), mat_spec, mat_spec],
        out_specs=mat_spec,
        out_shape=jax.ShapeDtypeStruct((N_PAD, D), jnp.float32),
    )(beta, h0, h1)


def kernel(feat_dst, feat_src0, feat_src1, edge_index0, edge_index1,
           attn_l0, attn_r0, attn_l1, attn_r1, fc_W, fc_b, sem_attn):
    el0, er0, el1, er1, mx = _logits_call(
        feat_dst, feat_src0, feat_src1, attn_l0, attn_r0, attn_l1, attn_r1)

    src0 = edge_index0[0].reshape(N_TILES, EDGES_PER_TILE)
    dst0t = edge_index0[1].reshape(N_TILES, N_SUBS, SUB)
    src1 = edge_index1[0].reshape(N_TILES, EDGES_PER_TILE)
    dst1t = edge_index1[1].reshape(N_TILES, N_SUBS, SUB)

    out0, out1, den0, den1 = _sc_call(
        feat_src0, feat_src1, src0, dst0t, src1, dst1t,
        el0.reshape(-1), er0.reshape(-1), el1.reshape(-1), er1.reshape(-1),
        mx)

    h0, h1, s0a, s1a = _combine1_call(
        out0, den0, out1, den1, fc_W, fc_b.reshape(1, D))

    mean0 = s0a[0] / N_DST
    mean1 = s1a[0] / N_DST
    wv = jnp.stack([jnp.dot(mean0, sem_attn[0]), jnp.dot(mean1, sem_attn[0])])
    beta = jax.nn.softmax(wv)

    z = _combine2_call(h0, h1, beta.reshape(1, 2))
    return z[:N_DST]
